# Initial kernel scaffold; baseline (speedup 1.0000x reference)
import jax
import jax.numpy as jnp
from jax.experimental import pallas as pl


def kernel(x, edge_index, edge_weight, W1, b1, g1, be1, W2, b2, g2, be2, W3, b3, g3, be3, Wl1, bl1, Wl2, bl2):
    raise NotImplementedError("write your pallas kernel here")


# speedup vs baseline: 8.3970x; 8.3970x over previous
import functools

import jax
import jax.numpy as jnp
from jax import lax
from jax.experimental import pallas as pl
from jax.experimental.pallas import tpu as pltpu
from jax.experimental.pallas import tpu_sc as plsc

N = 10000
E = 320000
D = 128
NC = 2
NS = 16
NW = NC * NS
EPT = E // NW
CH = 80
NCHUNK = EPT // CH
RPT = N // NS

_mesh = plsc.VectorSubcoreMesh(core_axis_name="c", subcore_axis_name="s")


@functools.partial(
    pl.kernel,
    out_type=jax.ShapeDtypeStruct((NC * N,), jnp.float32),
    mesh=_mesh,
    scratch_types=[
        pltpu.VMEM((CH,), jnp.int32),
        pltpu.VMEM((CH,), jnp.float32),
        pltpu.VMEM((1000,), jnp.float32),
        pltpu.VMEM_SHARED((N,), jnp.float32),
    ],
)
def _deg_kernel(dst_hbm, ew_hbm, zeros_hbm, deg_out, idx_v, w_v, stage, deg_sh):
    c = lax.axis_index("c")
    s = lax.axis_index("s")
    wid = s * NC + c

    @pl.when(s < 10)
    def _():
        pltpu.sync_copy(zeros_hbm.at[pl.ds(s * 1000, 1000)], stage)
        pltpu.sync_copy(stage, deg_sh.at[pl.ds(s * 1000, 1000)])

    plsc.subcore_barrier()

    def chunk(k, carry):
        base = wid * EPT + k * CH
        pltpu.sync_copy(dst_hbm.at[pl.ds(base, CH)], idx_v)
        pltpu.sync_copy(ew_hbm.at[pl.ds(base, CH)], w_v)
        pltpu.sync_copy(w_v, deg_sh.at[idx_v], add=True)
        return carry

    lax.fori_loop(0, NCHUNK, chunk, 0)
    plsc.subcore_barrier()

    @pl.when(s < 10)
    def _():
        pltpu.sync_copy(deg_sh.at[pl.ds(s * 1000, 1000)], stage)
        pltpu.sync_copy(stage, deg_out.at[pl.ds(c * N + s * 1000, 1000)])


@functools.partial(
    pl.kernel,
    out_type=jax.ShapeDtypeStruct((NC, N, D), jnp.float32),
    mesh=_mesh,
    scratch_types=[
        pltpu.VMEM((CH,), jnp.int32),
        pltpu.VMEM((CH,), jnp.int32),
        pltpu.VMEM((CH,), jnp.float32),
        pltpu.VMEM((CH, D), jnp.float32),
        pltpu.VMEM((CH, D), jnp.float32),
        pltpu.VMEM_SHARED((N, D), jnp.float32),
        pltpu.SemaphoreType.DMA,
    ],
)
def _agg_kernel(y_hbm, src_hbm, dst_hbm, ew_hbm, zeros_hbm, acc_out,
                idx_s, idx_d, w_v, rows, stage, acc_sh, gsem):
    c = lax.axis_index("c")
    s = lax.axis_index("s")
    wid = s * NC + c

    for i in range(8):
        r0 = 640 * s + 80 * i
        ok = r0 < N

        @pl.when(ok & (c == 0))
        def _():
            pltpu.sync_copy(y_hbm.at[pl.ds(r0, 80)], stage)

        @pl.when(ok & (c != 0))
        def _():
            pltpu.sync_copy(zeros_hbm.at[pl.ds(r0, 80)], stage)

        @pl.when(ok)
        def _():
            pltpu.sync_copy(stage, acc_sh.at[pl.ds(r0, 80)])

    plsc.subcore_barrier()

    def chunk(k, carry):
        base = wid * EPT + k * CH
        pltpu.sync_copy(src_hbm.at[pl.ds(base, CH)], idx_s)
        pltpu.sync_copy(dst_hbm.at[pl.ds(base, CH)], idx_d)
        pltpu.sync_copy(ew_hbm.at[pl.ds(base, CH)], w_v)
        pltpu.async_copy(y_hbm.at[idx_s], rows, gsem).wait()

        for g in range(CH // 16):
            wvec = w_v[pl.ds(g * 16, 16)]
            for t in range(16):
                e = g * 16 + t
                bc = wvec[t]
                for j in range(D // 16):
                    rows[e, pl.ds(j * 16, 16)] = rows[e, pl.ds(j * 16, 16)] * bc
        pltpu.sync_copy(rows, acc_sh.at[idx_d], add=True)
        return carry

    lax.fori_loop(0, NCHUNK, chunk, 0)
    plsc.subcore_barrier()
    for i in range(8):
        r0 = 640 * s + 80 * i

        @pl.when(r0 < N)
        def _():
            pltpu.sync_copy(acc_sh.at[pl.ds(r0, 80)], stage)
            pltpu.sync_copy(stage, acc_out.at[c, pl.ds(r0, 80)])


def _prep_body(x_ref, w_ref, dinv_ref, y_ref):
    y_ref[...] = jnp.dot(x_ref[...], w_ref[...],
                         preferred_element_type=jnp.float32) * dinv_ref[...]


_prep = pl.pallas_call(
    _prep_body,
    out_shape=jax.ShapeDtypeStruct((N, D), jnp.float32),
)


def _bn_lrelu(t, g, be):
    mu = jnp.mean(t, axis=0, keepdims=True)
    d = t - mu
    var = jnp.mean(d * d, axis=0, keepdims=True)
    h = d * lax.rsqrt(var + 1e-5) * g + be
    return jnp.where(h > 0, h, 0.1 * h)


def _mid_body(acc_ref, dinv_ref, b_ref, g_ref, be_ref, wn_ref, yn_ref):
    t = (acc_ref[0] + acc_ref[1]) * dinv_ref[...] + b_ref[...]
    h = _bn_lrelu(t, g_ref[...], be_ref[...])
    yn_ref[...] = jnp.dot(h, wn_ref[...],
                          preferred_element_type=jnp.float32) * dinv_ref[...]


_mid = pl.pallas_call(
    _mid_body,
    out_shape=jax.ShapeDtypeStruct((N, D), jnp.float32),
)


def _head_body(acc_ref, dinv_ref, b_ref, g_ref, be_ref,
               wl1_ref, bl1_ref, wl2_ref, bl2_ref, o_ref):
    t = (acc_ref[0] + acc_ref[1]) * dinv_ref[...] + b_ref[...]
    h = _bn_lrelu(t, g_ref[...], be_ref[...])
    z = jnp.dot(h, wl1_ref[...], preferred_element_type=jnp.float32) + bl1_ref[...]
    z = jnp.where(z > 0, z, 0.1 * z)
    o_ref[...] = jnp.dot(z, wl2_ref[...],
                         preferred_element_type=jnp.float32) + bl2_ref[...]


_head = pl.pallas_call(
    _head_body,
    out_shape=jax.ShapeDtypeStruct((N, 1), jnp.float32),
)


def kernel(x, edge_index, edge_weight, W1, b1, g1, be1, W2, b2, g2, be2,
           W3, b3, g3, be3, Wl1, bl1, Wl2, bl2):
    src = edge_index[0]
    dst = edge_index[1]
    zeros_nd = jnp.zeros((N, D), jnp.float32)
    zeros_n = jnp.zeros((N,), jnp.float32)

    degp = _deg_kernel(dst, edge_weight, zeros_n)
    dinv = lax.rsqrt(degp[:N] + degp[N:] + 1.0)[:, None]

    y1 = _prep(x, W1, dinv)
    acc1 = _agg_kernel(y1, src, dst, edge_weight, zeros_nd)
    y2 = _mid(acc1, dinv, b1.reshape(1, D), g1.reshape(1, D),
              be1.reshape(1, D), W2)
    acc2 = _agg_kernel(y2, src, dst, edge_weight, zeros_nd)
    y3 = _mid(acc2, dinv, b2.reshape(1, D), g2.reshape(1, D),
              be2.reshape(1, D), W3)
    acc3 = _agg_kernel(y3, src, dst, edge_weight, zeros_nd)
    out = _head(acc3, dinv, b3.reshape(1, D), g3.reshape(1, D),
                be3.reshape(1, D), Wl1, bl1.reshape(1, D),
                Wl2, bl2.reshape(1, 1))
    return out


# speedup vs baseline: 19.1590x; 2.2817x over previous
import functools

import jax
import jax.numpy as jnp
from jax import lax
from jax.experimental import pallas as pl
from jax.experimental.pallas import tpu as pltpu
from jax.experimental.pallas import tpu_sc as plsc

N = 10000
E = 320000
D = 128
NC = 2
NS = 16
NW = NC * NS
CH = 64
NCHUNK = 160
EPT = NCHUNK * CH
E_PAD = NW * EPT
RPT = N // NS

_mesh = plsc.VectorSubcoreMesh(core_axis_name="c", subcore_axis_name="s")


@functools.partial(
    pl.kernel,
    out_type=jax.ShapeDtypeStruct((NC * N,), jnp.float32),
    mesh=_mesh,
    scratch_types=[
        pltpu.VMEM((EPT,), jnp.int32),
        pltpu.VMEM((EPT,), jnp.float32),
        pltpu.VMEM((CH,), jnp.int32),
        pltpu.VMEM((CH,), jnp.float32),
        pltpu.VMEM((1000,), jnp.float32),
        pltpu.VMEM_SHARED((N,), jnp.float32),
    ],
)
def _deg_kernel(dst_hbm, ew_hbm, zeros_hbm, deg_out, idx_d, w_v, idx_1d, w_1d,
                stage, deg_sh):
    c = lax.axis_index("c")
    s = lax.axis_index("s")
    wid = s * NC + c

    @pl.when(s < 10)
    def _():
        pltpu.sync_copy(zeros_hbm.at[pl.ds(s * 1000, 1000)], stage)
        pltpu.sync_copy(stage, deg_sh.at[pl.ds(s * 1000, 1000)])

    pltpu.sync_copy(dst_hbm.at[pl.ds(wid * EPT, EPT)], idx_d)
    pltpu.sync_copy(ew_hbm.at[pl.ds(wid * EPT, EPT)], w_v)
    plsc.subcore_barrier()

    def chunk(k, carry):
        for g in range(CH // 16):
            idx_1d[pl.ds(g * 16, 16)] = idx_d[pl.ds(k * CH + g * 16, 16)]
            w_1d[pl.ds(g * 16, 16)] = w_v[pl.ds(k * CH + g * 16, 16)]
        pltpu.sync_copy(w_1d, deg_sh.at[idx_1d], add=True)
        return carry

    lax.fori_loop(0, NCHUNK, chunk, 0)
    plsc.subcore_barrier()

    @pl.when(s < 10)
    def _():
        pltpu.sync_copy(deg_sh.at[pl.ds(s * 1000, 1000)], stage)
        pltpu.sync_copy(stage, deg_out.at[pl.ds(c * N + s * 1000, 1000)])


@functools.partial(
    pl.kernel,
    out_type=jax.ShapeDtypeStruct((NC, N, D), jnp.float32),
    mesh=_mesh,
    scratch_types=[
        pltpu.VMEM((EPT // 4,), jnp.int32),
        pltpu.VMEM((EPT // 4,), jnp.int32),
        pltpu.VMEM((EPT // 4,), jnp.float32),
        pltpu.VMEM((CH, D), jnp.float32),
        pltpu.VMEM((CH, D), jnp.float32),
        pltpu.VMEM((CH,), jnp.int32),
        pltpu.VMEM((CH,), jnp.int32),
        pltpu.VMEM_SHARED((N, D), jnp.float32),
        pltpu.SemaphoreType.DMA,
        pltpu.SemaphoreType.DMA,
    ],
)
def _agg_kernel(y_hbm, src_hbm, dst_hbm, ew_hbm, zeros_hbm, acc_out,
                idx_s, idx_d, w_v, rows0, rows1, idx0_1d, idx1_1d, acc_sh,
                gsem0, gsem1):
    c = lax.axis_index("c")
    s = lax.axis_index("s")
    wid = s * NC + c

    for i in range(10):
        r0 = 640 * s + 64 * i
        ok = r0 + 64 <= N

        @pl.when(ok & (c == 0))
        def _():
            pltpu.sync_copy(y_hbm.at[pl.ds(r0, 64)], rows0)

        @pl.when(ok & (c != 0))
        def _():
            pltpu.sync_copy(zeros_hbm.at[pl.ds(r0, 64)], rows0)

        @pl.when(ok)
        def _():
            pltpu.sync_copy(rows0, acc_sh.at[pl.ds(r0, 64)])

    TAIL0 = N - N % 64

    @pl.when((s == NS - 1) & (c == 0))
    def _():
        pltpu.sync_copy(y_hbm.at[pl.ds(TAIL0, N % 64)],
                        rows0.at[pl.ds(0, N % 64)])

    @pl.when((s == NS - 1) & (c != 0))
    def _():
        pltpu.sync_copy(zeros_hbm.at[pl.ds(TAIL0, N % 64)],
                        rows0.at[pl.ds(0, N % 64)])

    @pl.when(s == NS - 1)
    def _():
        pltpu.sync_copy(rows0.at[pl.ds(0, N % 64)],
                        acc_sh.at[pl.ds(TAIL0, N % 64)])

    plsc.subcore_barrier()

    def _scale(rows, k):
        def grp(g, carry):
            wvec = w_v[pl.ds(k * CH + g * 16, 16)]
            for t in range(16):
                e = g * 16 + t
                bc = wvec[t]
                for j in range(D // 16):
                    rows[e, pl.ds(j * 16, 16)] = rows[e, pl.ds(j * 16, 16)] * bc
            return carry

        lax.fori_loop(0, CH // 16, grp, 0)

    bufs = ((rows0, gsem0, idx0_1d), (rows1, gsem1, idx1_1d))
    Q = NCHUNK // 4
    QE = EPT // 4
    for q in range(4):
        pltpu.sync_copy(src_hbm.at[pl.ds(wid * EPT + q * QE, QE)], idx_s)
        pltpu.sync_copy(dst_hbm.at[pl.ds(wid * EPT + q * QE, QE)], idx_d)
        pltpu.sync_copy(ew_hbm.at[pl.ds(wid * EPT + q * QE, QE)], w_v)
        pltpu.async_copy(y_hbm.at[idx_s.at[pl.ds(0, CH)]], rows0, gsem0)
        pltpu.async_copy(y_hbm.at[idx_s.at[pl.ds(CH, CH)]], rows1, gsem1)

        def step(i, carry):
            for b in range(2):
                k = 2 * i + b
                rows, gsem, idx_1d = bufs[b]
                for g in range(CH // 16):
                    idx_1d[pl.ds(g * 16, 16)] = idx_d[pl.ds(k * CH + g * 16, 16)]
                pltpu.make_async_copy(
                    y_hbm.at[idx_s.at[pl.ds(k * CH, CH)]], rows, gsem
                ).wait()
                _scale(rows, k)
                pltpu.sync_copy(rows, acc_sh.at[idx_1d], add=True)

                @pl.when(k + 2 < Q)
                def _():
                    pltpu.async_copy(
                        y_hbm.at[idx_s.at[pl.ds((k + 2) * CH, CH)]], rows, gsem
                    )

            return carry

        lax.fori_loop(0, Q // 2, step, 0)

    plsc.subcore_barrier()
    for i in range(10):
        r0 = 640 * s + 64 * i

        @pl.when(r0 + 64 <= N)
        def _():
            pltpu.sync_copy(acc_sh.at[pl.ds(r0, 64)], rows0)
            pltpu.sync_copy(rows0, acc_out.at[c, pl.ds(r0, 64)])

    @pl.when(s == NS - 1)
    def _():
        pltpu.sync_copy(acc_sh.at[pl.ds(TAIL0, N % 64)],
                        rows0.at[pl.ds(0, N % 64)])
        pltpu.sync_copy(rows0.at[pl.ds(0, N % 64)],
                        acc_out.at[c, pl.ds(TAIL0, N % 64)])


def _prep_body(x_ref, w_ref, dinv_ref, y_ref):
    y_ref[...] = jnp.dot(x_ref[...], w_ref[...],
                         preferred_element_type=jnp.float32) * dinv_ref[...]


_prep = pl.pallas_call(
    _prep_body,
    out_shape=jax.ShapeDtypeStruct((N, D), jnp.float32),
)


def _bn_lrelu(t, g, be):
    mu = jnp.mean(t, axis=0, keepdims=True)
    d = t - mu
    var = jnp.mean(d * d, axis=0, keepdims=True)
    h = d * lax.rsqrt(var + 1e-5) * g + be
    return jnp.where(h > 0, h, 0.1 * h)


def _mid_body(acc_ref, dinv_ref, b_ref, g_ref, be_ref, wn_ref, yn_ref):
    t = (acc_ref[0] + acc_ref[1]) * dinv_ref[...] + b_ref[...]
    h = _bn_lrelu(t, g_ref[...], be_ref[...])
    yn_ref[...] = jnp.dot(h, wn_ref[...],
                          preferred_element_type=jnp.float32) * dinv_ref[...]


_mid = pl.pallas_call(
    _mid_body,
    out_shape=jax.ShapeDtypeStruct((N, D), jnp.float32),
)


def _head_body(acc_ref, dinv_ref, b_ref, g_ref, be_ref,
               wl1_ref, bl1_ref, wl2_ref, bl2_ref, o_ref):
    t = (acc_ref[0] + acc_ref[1]) * dinv_ref[...] + b_ref[...]
    h = _bn_lrelu(t, g_ref[...], be_ref[...])
    z = jnp.dot(h, wl1_ref[...], preferred_element_type=jnp.float32) + bl1_ref[...]
    z = jnp.where(z > 0, z, 0.1 * z)
    o_ref[...] = jnp.dot(z, wl2_ref[...],
                         preferred_element_type=jnp.float32) + bl2_ref[...]


_head = pl.pallas_call(
    _head_body,
    out_shape=jax.ShapeDtypeStruct((N, 1), jnp.float32),
)


def kernel(x, edge_index, edge_weight, W1, b1, g1, be1, W2, b2, g2, be2,
           W3, b3, g3, be3, Wl1, bl1, Wl2, bl2):
    pad_idx = jnp.arange(E_PAD - E, dtype=jnp.int32) % N
    src = jnp.concatenate([edge_index[0], pad_idx])
    dst = jnp.concatenate([edge_index[1], pad_idx])
    ew = jnp.concatenate([edge_weight, jnp.zeros((E_PAD - E,), jnp.float32)])
    zeros_nd = jnp.zeros((N, D), jnp.float32)
    zeros_n = jnp.zeros((N,), jnp.float32)

    degp = _deg_kernel(dst, ew, zeros_n)
    dinv = lax.rsqrt(degp[:N] + degp[N:] + 1.0)[:, None]

    y1 = _prep(x, W1, dinv)
    acc1 = _agg_kernel(y1, src, dst, ew, zeros_nd)
    y2 = _mid(acc1, dinv, b1.reshape(1, D), g1.reshape(1, D),
              be1.reshape(1, D), W2)
    acc2 = _agg_kernel(y2, src, dst, ew, zeros_nd)
    y3 = _mid(acc2, dinv, b2.reshape(1, D), g2.reshape(1, D),
              be2.reshape(1, D), W3)
    acc3 = _agg_kernel(y3, src, dst, ew, zeros_nd)
    out = _head(acc3, dinv, b3.reshape(1, D), g3.reshape(1, D),
                be3.reshape(1, D), Wl1, bl1.reshape(1, D),
                Wl2, bl2.reshape(1, 1))
    return out


# speedup vs baseline: 21.1496x; 1.1039x over previous
import functools

import jax
import jax.numpy as jnp
from jax import lax
from jax.experimental import pallas as pl
from jax.experimental.pallas import tpu as pltpu
from jax.experimental.pallas import tpu_sc as plsc

N = 10000
E = 320000
D = 128
NC = 2
NS = 16
NW = NC * NS
CH = 64
NCHUNK = 160
EPT = NCHUNK * CH
E_PAD = NW * EPT
RPT = N // NS

_mesh = plsc.VectorSubcoreMesh(core_axis_name="c", subcore_axis_name="s")


@functools.partial(
    pl.kernel,
    out_type=jax.ShapeDtypeStruct((NC * N,), jnp.float32),
    mesh=_mesh,
    scratch_types=[
        pltpu.VMEM((EPT,), jnp.int32),
        pltpu.VMEM((EPT,), jnp.float32),
        pltpu.VMEM((CH,), jnp.int32),
        pltpu.VMEM((CH,), jnp.float32),
        pltpu.VMEM((1000,), jnp.float32),
        pltpu.VMEM_SHARED((N,), jnp.float32),
    ],
)
def _deg_kernel(dst_hbm, ew_hbm, zeros_hbm, deg_out, idx_d, w_v, idx_1d, w_1d,
                stage, deg_sh):
    c = lax.axis_index("c")
    s = lax.axis_index("s")
    wid = s * NC + c

    @pl.when(s < 10)
    def _():
        pltpu.sync_copy(zeros_hbm.at[pl.ds(s * 1000, 1000)], stage)
        pltpu.sync_copy(stage, deg_sh.at[pl.ds(s * 1000, 1000)])

    pltpu.sync_copy(dst_hbm.at[pl.ds(wid * EPT, EPT)], idx_d)
    pltpu.sync_copy(ew_hbm.at[pl.ds(wid * EPT, EPT)], w_v)
    plsc.subcore_barrier()

    def chunk(k, carry):
        for g in range(CH // 16):
            idx_1d[pl.ds(g * 16, 16)] = idx_d[pl.ds(k * CH + g * 16, 16)]
            w_1d[pl.ds(g * 16, 16)] = w_v[pl.ds(k * CH + g * 16, 16)]
        pltpu.sync_copy(w_1d, deg_sh.at[idx_1d], add=True)
        return carry

    lax.fori_loop(0, NCHUNK, chunk, 0)
    plsc.subcore_barrier()

    @pl.when(s < 10)
    def _():
        pltpu.sync_copy(deg_sh.at[pl.ds(s * 1000, 1000)], stage)
        pltpu.sync_copy(stage, deg_out.at[pl.ds(c * N + s * 1000, 1000)])


@functools.partial(
    pl.kernel,
    out_type=jax.ShapeDtypeStruct((NC, N, D), jnp.float32),
    mesh=_mesh,
    scratch_types=[
        pltpu.VMEM((EPT // 4,), jnp.int32),
        pltpu.VMEM((EPT // 4,), jnp.int32),
        pltpu.VMEM((EPT // 4,), jnp.float32),
        pltpu.VMEM((CH, D), jnp.float32),
        pltpu.VMEM((CH, D), jnp.float32),
        pltpu.VMEM((CH, D), jnp.float32),
        pltpu.VMEM((CH,), jnp.int32),
        pltpu.VMEM((CH,), jnp.int32),
        pltpu.VMEM((CH,), jnp.int32),
        pltpu.VMEM_SHARED((N, D), jnp.float32),
        pltpu.SemaphoreType.DMA,
        pltpu.SemaphoreType.DMA,
        pltpu.SemaphoreType.DMA,
        pltpu.SemaphoreType.DMA,
        pltpu.SemaphoreType.DMA,
        pltpu.SemaphoreType.DMA,
    ],
)
def _agg_kernel(y_hbm, src_hbm, dst_hbm, ew_hbm, zeros_hbm, acc_out,
                idx_s, idx_d, w_v, rows0, rows1, rows2, idx0_1d, idx1_1d,
                idx2_1d, acc_sh, gsem0, gsem1, gsem2, ssem0, ssem1, ssem2):
    c = lax.axis_index("c")
    s = lax.axis_index("s")
    wid = s * NC + c

    for i in range(10):
        r0 = 640 * s + 64 * i
        ok = r0 + 64 <= N

        @pl.when(ok & (c == 0))
        def _():
            pltpu.sync_copy(y_hbm.at[pl.ds(r0, 64)], rows0)

        @pl.when(ok & (c != 0))
        def _():
            pltpu.sync_copy(zeros_hbm.at[pl.ds(r0, 64)], rows0)

        @pl.when(ok)
        def _():
            pltpu.sync_copy(rows0, acc_sh.at[pl.ds(r0, 64)])

    TAIL0 = N - N % 64

    @pl.when((s == NS - 1) & (c == 0))
    def _():
        pltpu.sync_copy(y_hbm.at[pl.ds(TAIL0, N % 64)],
                        rows0.at[pl.ds(0, N % 64)])

    @pl.when((s == NS - 1) & (c != 0))
    def _():
        pltpu.sync_copy(zeros_hbm.at[pl.ds(TAIL0, N % 64)],
                        rows0.at[pl.ds(0, N % 64)])

    @pl.when(s == NS - 1)
    def _():
        pltpu.sync_copy(rows0.at[pl.ds(0, N % 64)],
                        acc_sh.at[pl.ds(TAIL0, N % 64)])

    plsc.subcore_barrier()

    def _scale(rows, k):
        def grp(g, carry):
            wvec = w_v[pl.ds(k * CH + g * 16, 16)]
            for t in range(16):
                e = g * 16 + t
                bc = wvec[t]
                for j in range(D // 16):
                    rows[e, pl.ds(j * 16, 16)] = rows[e, pl.ds(j * 16, 16)] * bc
            return carry

        lax.fori_loop(0, CH // 16, grp, 0)

    bufs = ((rows0, gsem0, ssem0, idx0_1d), (rows1, gsem1, ssem1, idx1_1d),
            (rows2, gsem2, ssem2, idx2_1d))
    Q = NCHUNK // 4
    QE = EPT // 4

    def _gather(k, b):
        return pltpu.make_async_copy(
            y_hbm.at[idx_s.at[pl.ds(k * CH, CH)]], bufs[b][0], bufs[b][1])

    def _scatter(b):
        return pltpu.make_async_copy(bufs[b][0], acc_sh.at[bufs[b][3]],
                                     bufs[b][2])

    def _chunk(k, b):
        rows, gsem, ssem, idx_1d = bufs[b]
        for g in range(CH // 16):
            idx_1d[pl.ds(g * 16, 16)] = idx_d[pl.ds(k * CH + g * 16, 16)]
        _gather(k, b).wait()
        _scale(rows, k)
        _scatter(b).start(add=True)

    for q in range(4):
        pltpu.sync_copy(src_hbm.at[pl.ds(wid * EPT + q * QE, QE)], idx_s)
        pltpu.sync_copy(dst_hbm.at[pl.ds(wid * EPT + q * QE, QE)], idx_d)
        pltpu.sync_copy(ew_hbm.at[pl.ds(wid * EPT + q * QE, QE)], w_v)
        _gather(0, 0).start()
        _gather(1, 1).start()

        def step(i, carry):
            for b in range(3):
                k = 3 * i + b
                _chunk(k, b)
                kp = k + 2
                bp = (b + 2) % 3

                @pl.when(kp < Q)
                def _():
                    @pl.when(k >= 1)
                    def _():
                        _scatter(bp).wait()

                    _gather(kp, bp).start()

            return carry

        lax.fori_loop(0, Q // 3, step, 0)
        _chunk(Q - 1, (Q - 1) % 3)
        for b in range(3):
            _scatter(b).wait()

    plsc.subcore_barrier()
    for i in range(10):
        r0 = 640 * s + 64 * i

        @pl.when(r0 + 64 <= N)
        def _():
            pltpu.sync_copy(acc_sh.at[pl.ds(r0, 64)], rows0)
            pltpu.sync_copy(rows0, acc_out.at[c, pl.ds(r0, 64)])

    @pl.when(s == NS - 1)
    def _():
        pltpu.sync_copy(acc_sh.at[pl.ds(TAIL0, N % 64)],
                        rows0.at[pl.ds(0, N % 64)])
        pltpu.sync_copy(rows0.at[pl.ds(0, N % 64)],
                        acc_out.at[c, pl.ds(TAIL0, N % 64)])


def _prep_body(x_ref, w_ref, dinv_ref, y_ref):
    y_ref[...] = jnp.dot(x_ref[...], w_ref[...],
                         preferred_element_type=jnp.float32) * dinv_ref[...]


_prep = pl.pallas_call(
    _prep_body,
    out_shape=jax.ShapeDtypeStruct((N, D), jnp.float32),
)


def _bn_lrelu(t, g, be):
    mu = jnp.mean(t, axis=0, keepdims=True)
    d = t - mu
    var = jnp.mean(d * d, axis=0, keepdims=True)
    h = d * lax.rsqrt(var + 1e-5) * g + be
    return jnp.where(h > 0, h, 0.1 * h)


def _mid_body(acc_ref, dinv_ref, b_ref, g_ref, be_ref, wn_ref, yn_ref):
    t = (acc_ref[0] + acc_ref[1]) * dinv_ref[...] + b_ref[...]
    h = _bn_lrelu(t, g_ref[...], be_ref[...])
    yn_ref[...] = jnp.dot(h, wn_ref[...],
                          preferred_element_type=jnp.float32) * dinv_ref[...]


_mid = pl.pallas_call(
    _mid_body,
    out_shape=jax.ShapeDtypeStruct((N, D), jnp.float32),
)


def _head_body(acc_ref, dinv_ref, b_ref, g_ref, be_ref,
               wl1_ref, bl1_ref, wl2_ref, bl2_ref, o_ref):
    t = (acc_ref[0] + acc_ref[1]) * dinv_ref[...] + b_ref[...]
    h = _bn_lrelu(t, g_ref[...], be_ref[...])
    z = jnp.dot(h, wl1_ref[...], preferred_element_type=jnp.float32) + bl1_ref[...]
    z = jnp.where(z > 0, z, 0.1 * z)
    o_ref[...] = jnp.dot(z, wl2_ref[...],
                         preferred_element_type=jnp.float32) + bl2_ref[...]


_head = pl.pallas_call(
    _head_body,
    out_shape=jax.ShapeDtypeStruct((N, 1), jnp.float32),
)


def kernel(x, edge_index, edge_weight, W1, b1, g1, be1, W2, b2, g2, be2,
           W3, b3, g3, be3, Wl1, bl1, Wl2, bl2):
    pad_idx = jnp.arange(E_PAD - E, dtype=jnp.int32) % N
    src = jnp.concatenate([edge_index[0], pad_idx])
    dst = jnp.concatenate([edge_index[1], pad_idx])
    ew = jnp.concatenate([edge_weight, jnp.zeros((E_PAD - E,), jnp.float32)])
    zeros_nd = jnp.zeros((N, D), jnp.float32)
    zeros_n = jnp.zeros((N,), jnp.float32)

    degp = _deg_kernel(dst, ew, zeros_n)
    dinv = lax.rsqrt(degp[:N] + degp[N:] + 1.0)[:, None]

    y1 = _prep(x, W1, dinv)
    acc1 = _agg_kernel(y1, src, dst, ew, zeros_nd)
    y2 = _mid(acc1, dinv, b1.reshape(1, D), g1.reshape(1, D),
              be1.reshape(1, D), W2)
    acc2 = _agg_kernel(y2, src, dst, ew, zeros_nd)
    y3 = _mid(acc2, dinv, b2.reshape(1, D), g2.reshape(1, D),
              be2.reshape(1, D), W3)
    acc3 = _agg_kernel(y3, src, dst, ew, zeros_nd)
    out = _head(acc3, dinv, b3.reshape(1, D), g3.reshape(1, D),
                be3.reshape(1, D), Wl1, bl1.reshape(1, D),
                Wl2, bl2.reshape(1, 1))
    return out


# speedup vs baseline: 21.7055x; 1.0263x over previous
import functools

import jax
import jax.numpy as jnp
from jax import lax
from jax.experimental import pallas as pl
from jax.experimental.pallas import tpu as pltpu
from jax.experimental.pallas import tpu_sc as plsc

N = 10000
E = 320000
D = 128
NC = 2
NS = 16
NW = NC * NS
CH = 80
NCHUNK = 128
EPT = NCHUNK * CH
E_PAD = NW * EPT
RPT = N // NS

_mesh = plsc.VectorSubcoreMesh(core_axis_name="c", subcore_axis_name="s")


@functools.partial(
    pl.kernel,
    out_type=jax.ShapeDtypeStruct((NC * N,), jnp.float32),
    mesh=_mesh,
    scratch_types=[
        pltpu.VMEM((EPT,), jnp.int32),
        pltpu.VMEM((EPT,), jnp.float32),
        pltpu.VMEM((CH,), jnp.int32),
        pltpu.VMEM((CH,), jnp.float32),
        pltpu.VMEM((1000,), jnp.float32),
        pltpu.VMEM_SHARED((N,), jnp.float32),
    ],
)
def _deg_kernel(dst_hbm, ew_hbm, zeros_hbm, deg_out, idx_d, w_v, idx_1d, w_1d,
                stage, deg_sh):
    c = lax.axis_index("c")
    s = lax.axis_index("s")
    wid = s * NC + c

    @pl.when(s < 10)
    def _():
        pltpu.sync_copy(zeros_hbm.at[pl.ds(s * 1000, 1000)], stage)
        pltpu.sync_copy(stage, deg_sh.at[pl.ds(s * 1000, 1000)])

    pltpu.sync_copy(dst_hbm.at[pl.ds(wid * EPT, EPT)], idx_d)
    pltpu.sync_copy(ew_hbm.at[pl.ds(wid * EPT, EPT)], w_v)
    plsc.subcore_barrier()

    def chunk(k, carry):
        for g in range(CH // 16):
            idx_1d[pl.ds(g * 16, 16)] = idx_d[pl.ds(k * CH + g * 16, 16)]
            w_1d[pl.ds(g * 16, 16)] = w_v[pl.ds(k * CH + g * 16, 16)]
        pltpu.sync_copy(w_1d, deg_sh.at[idx_1d], add=True)
        return carry

    lax.fori_loop(0, NCHUNK, chunk, 0)
    plsc.subcore_barrier()

    @pl.when(s < 10)
    def _():
        pltpu.sync_copy(deg_sh.at[pl.ds(s * 1000, 1000)], stage)
        pltpu.sync_copy(stage, deg_out.at[pl.ds(c * N + s * 1000, 1000)])


@functools.partial(
    pl.kernel,
    out_type=jax.ShapeDtypeStruct((NC, N, D), jnp.float32),
    mesh=_mesh,
    scratch_types=[
        pltpu.VMEM((EPT // 4,), jnp.int32),
        pltpu.VMEM((EPT // 4,), jnp.int32),
        pltpu.VMEM((EPT // 4,), jnp.float32),
        pltpu.VMEM((CH, D), jnp.float32),
        pltpu.VMEM((CH, D), jnp.float32),
        pltpu.VMEM((CH, D), jnp.float32),
        pltpu.VMEM((CH,), jnp.int32),
        pltpu.VMEM((CH,), jnp.int32),
        pltpu.VMEM((CH,), jnp.int32),
        pltpu.VMEM_SHARED((N, D), jnp.float32),
        pltpu.SemaphoreType.DMA,
        pltpu.SemaphoreType.DMA,
        pltpu.SemaphoreType.DMA,
        pltpu.SemaphoreType.DMA,
        pltpu.SemaphoreType.DMA,
        pltpu.SemaphoreType.DMA,
    ],
)
def _agg_kernel(y_hbm, src_hbm, dst_hbm, ew_hbm, zeros_hbm, acc_out,
                idx_s, idx_d, w_v, rows0, rows1, rows2, idx0_1d, idx1_1d,
                idx2_1d, acc_sh, gsem0, gsem1, gsem2, ssem0, ssem1, ssem2):
    c = lax.axis_index("c")
    s = lax.axis_index("s")
    wid = s * NC + c

    for i in range(10):
        r0 = 640 * s + 64 * i
        ok = r0 + 64 <= N

        @pl.when(ok & (c == 0))
        def _():
            pltpu.sync_copy(y_hbm.at[pl.ds(r0, 64)], rows0.at[pl.ds(0, 64)])

        @pl.when(ok & (c != 0))
        def _():
            pltpu.sync_copy(zeros_hbm.at[pl.ds(r0, 64)], rows0.at[pl.ds(0, 64)])

        @pl.when(ok)
        def _():
            pltpu.sync_copy(rows0.at[pl.ds(0, 64)], acc_sh.at[pl.ds(r0, 64)])

    TAIL0 = N - N % 64

    @pl.when((s == NS - 1) & (c == 0))
    def _():
        pltpu.sync_copy(y_hbm.at[pl.ds(TAIL0, N % 64)],
                        rows0.at[pl.ds(0, N % 64)])

    @pl.when((s == NS - 1) & (c != 0))
    def _():
        pltpu.sync_copy(zeros_hbm.at[pl.ds(TAIL0, N % 64)],
                        rows0.at[pl.ds(0, N % 64)])

    @pl.when(s == NS - 1)
    def _():
        pltpu.sync_copy(rows0.at[pl.ds(0, N % 64)],
                        acc_sh.at[pl.ds(TAIL0, N % 64)])

    plsc.subcore_barrier()

    def _scale(rows, k):
        def grp(g, carry):
            wvec = w_v[pl.ds(k * CH + g * 16, 16)]
            for t in range(16):
                e = g * 16 + t
                bc = wvec[t]
                for j in range(D // 16):
                    rows[e, pl.ds(j * 16, 16)] = rows[e, pl.ds(j * 16, 16)] * bc
            return carry

        lax.fori_loop(0, CH // 16, grp, 0)

    bufs = ((rows0, gsem0, ssem0, idx0_1d), (rows1, gsem1, ssem1, idx1_1d),
            (rows2, gsem2, ssem2, idx2_1d))
    Q = NCHUNK // 4
    QE = EPT // 4

    def _gather(k, b):
        return pltpu.make_async_copy(
            y_hbm.at[idx_s.at[pl.ds(k * CH, CH)]], bufs[b][0], bufs[b][1])

    def _scatter(b):
        return pltpu.make_async_copy(bufs[b][0], acc_sh.at[bufs[b][3]],
                                     bufs[b][2])

    def _chunk(k, b):
        rows, gsem, ssem, idx_1d = bufs[b]
        for g in range(CH // 16):
            idx_1d[pl.ds(g * 16, 16)] = idx_d[pl.ds(k * CH + g * 16, 16)]
        _gather(k, b).wait()
        _scale(rows, k)
        _scatter(b).start(add=True)

    for q in range(4):
        pltpu.sync_copy(src_hbm.at[pl.ds(wid * EPT + q * QE, QE)], idx_s)
        pltpu.sync_copy(dst_hbm.at[pl.ds(wid * EPT + q * QE, QE)], idx_d)
        pltpu.sync_copy(ew_hbm.at[pl.ds(wid * EPT + q * QE, QE)], w_v)
        _gather(0, 0).start()
        _gather(1, 1).start()

        def step(i, carry):
            for b in range(3):
                k = 3 * i + b
                _chunk(k, b)
                kp = k + 2
                bp = (b + 2) % 3

                @pl.when(kp < Q)
                def _():
                    @pl.when(k >= 1)
                    def _():
                        _scatter(bp).wait()

                    _gather(kp, bp).start()

            return carry

        lax.fori_loop(0, Q // 3, step, 0)
        for kk in range(3 * (Q // 3), Q):
            _chunk(kk, kk % 3)
        for b in range(3):
            _scatter(b).wait()

    plsc.subcore_barrier()
    for i in range(10):
        r0 = 640 * s + 64 * i

        @pl.when(r0 + 64 <= N)
        def _():
            pltpu.sync_copy(acc_sh.at[pl.ds(r0, 64)], rows0.at[pl.ds(0, 64)])
            pltpu.sync_copy(rows0.at[pl.ds(0, 64)], acc_out.at[c, pl.ds(r0, 64)])

    @pl.when(s == NS - 1)
    def _():
        pltpu.sync_copy(acc_sh.at[pl.ds(TAIL0, N % 64)],
                        rows0.at[pl.ds(0, N % 64)])
        pltpu.sync_copy(rows0.at[pl.ds(0, N % 64)],
                        acc_out.at[c, pl.ds(TAIL0, N % 64)])


def _prep_body(x_ref, w_ref, dinv_ref, y_ref):
    y_ref[...] = jnp.dot(x_ref[...], w_ref[...],
                         preferred_element_type=jnp.float32) * dinv_ref[...]


_prep = pl.pallas_call(
    _prep_body,
    out_shape=jax.ShapeDtypeStruct((N, D), jnp.float32),
)


def _bn_lrelu(t, g, be):
    mu = jnp.mean(t, axis=0, keepdims=True)
    d = t - mu
    var = jnp.mean(d * d, axis=0, keepdims=True)
    h = d * lax.rsqrt(var + 1e-5) * g + be
    return jnp.where(h > 0, h, 0.1 * h)


def _mid_body(acc_ref, dinv_ref, b_ref, g_ref, be_ref, wn_ref, yn_ref):
    t = (acc_ref[0] + acc_ref[1]) * dinv_ref[...] + b_ref[...]
    h = _bn_lrelu(t, g_ref[...], be_ref[...])
    yn_ref[...] = jnp.dot(h, wn_ref[...],
                          preferred_element_type=jnp.float32) * dinv_ref[...]


_mid = pl.pallas_call(
    _mid_body,
    out_shape=jax.ShapeDtypeStruct((N, D), jnp.float32),
)


def _head_body(acc_ref, dinv_ref, b_ref, g_ref, be_ref,
               wl1_ref, bl1_ref, wl2_ref, bl2_ref, o_ref):
    t = (acc_ref[0] + acc_ref[1]) * dinv_ref[...] + b_ref[...]
    h = _bn_lrelu(t, g_ref[...], be_ref[...])
    z = jnp.dot(h, wl1_ref[...], preferred_element_type=jnp.float32) + bl1_ref[...]
    z = jnp.where(z > 0, z, 0.1 * z)
    o_ref[...] = jnp.dot(z, wl2_ref[...],
                         preferred_element_type=jnp.float32) + bl2_ref[...]


_head = pl.pallas_call(
    _head_body,
    out_shape=jax.ShapeDtypeStruct((N, 1), jnp.float32),
)


def kernel(x, edge_index, edge_weight, W1, b1, g1, be1, W2, b2, g2, be2,
           W3, b3, g3, be3, Wl1, bl1, Wl2, bl2):
    pad_idx = jnp.arange(E_PAD - E, dtype=jnp.int32) % N
    src = jnp.concatenate([edge_index[0], pad_idx])
    dst = jnp.concatenate([edge_index[1], pad_idx])
    ew = jnp.concatenate([edge_weight, jnp.zeros((E_PAD - E,), jnp.float32)])
    zeros_nd = jnp.zeros((N, D), jnp.float32)
    zeros_n = jnp.zeros((N,), jnp.float32)

    degp = _deg_kernel(dst, ew, zeros_n)
    dinv = lax.rsqrt(degp[:N] + degp[N:] + 1.0)[:, None]

    y1 = _prep(x, W1, dinv)
    acc1 = _agg_kernel(y1, src, dst, ew, zeros_nd)
    y2 = _mid(acc1, dinv, b1.reshape(1, D), g1.reshape(1, D),
              be1.reshape(1, D), W2)
    acc2 = _agg_kernel(y2, src, dst, ew, zeros_nd)
    y3 = _mid(acc2, dinv, b2.reshape(1, D), g2.reshape(1, D),
              be2.reshape(1, D), W3)
    acc3 = _agg_kernel(y3, src, dst, ew, zeros_nd)
    out = _head(acc3, dinv, b3.reshape(1, D), g3.reshape(1, D),
                be3.reshape(1, D), Wl1, bl1.reshape(1, D),
                Wl2, bl2.reshape(1, 1))
    return out
